# baseline (speedup 1.0000x reference)
import functools

import jax
import jax.numpy as jnp
from jax import lax
from jax.experimental import pallas as pl
from jax.experimental.pallas import tpu as pltpu
from jax.experimental.pallas import tpu_sc as plsc

_N = 10000
_E = 320000
_G = 64
_C = 10
_RB = 1000
_NB = _N // _RB

_NTILES = 32
_BLK = 128
_RANGE = 320
_ACC_T = 328
_ACC_N = _NTILES * _RANGE
_NBT = (_E + _NTILES * _BLK) // _BLK

_BN_SCALE = 1.0 / (1.0 + 1e-5) ** 0.5


def _make_sc_layer():
    d = 128
    mesh = plsc.VectorSubcoreMesh(core_axis_name="c", subcore_axis_name="s")

    @functools.partial(
        pl.kernel,
        out_type=jax.ShapeDtypeStruct((_ACC_N, d), jnp.float32),
        mesh=mesh,
        scratch_types=[
            pltpu.VMEM((_ACC_T, d), jnp.float32),
            pltpu.VMEM((_BLK,), jnp.int32),
            pltpu.VMEM((_BLK,), jnp.int32),
            pltpu.VMEM((_BLK,), jnp.float32),
            pltpu.VMEM((_BLK, d), jnp.float32),
            pltpu.VMEM((2, d), jnp.float32),
            pltpu.VMEM((16,), jnp.int32),
            pltpu.SemaphoreType.DMA,
        ],
    )
    def sc_layer(z_hbm, src_hbm, dstl_hbm, ea_hbm, lwb_hbm, offs_hbm,
                 zeros_hbm, out_hbm,
                 acc, src_v, dstl_v, ea_v, rows, lwb, offs_v, gsem):
        cid = lax.axis_index("c")
        sid = lax.axis_index("s")
        w = cid * 16 + sid

        pltpu.sync_copy(lwb_hbm, lwb)
        pltpu.sync_copy(offs_hbm.at[w], offs_v)
        pltpu.sync_copy(zeros_hbm, acc)
        ov = offs_v[...]
        start = ov[0]
        nblk = ov[1]

        def block(b, carry):
            blk = start + b
            pltpu.sync_copy(src_hbm.at[blk], src_v)
            pltpu.sync_copy(dstl_hbm.at[blk], dstl_v)
            pltpu.sync_copy(ea_hbm.at[blk], ea_v)
            pltpu.async_copy(z_hbm.at[src_v], rows, gsem).wait()

            def row_group(rr, c2):
                eav = ea_v[pl.ds(rr * 16, 16)]
                dstv = dstl_v[pl.ds(rr * 16, 16)]
                for r16 in range(16):
                    r = rr * 16 + r16
                    eab = eav[r16]
                    dl = dstv[r16]
                    for c in range(d // 16):
                        sl = pl.ds(c * 16, 16)
                        m = jnp.maximum(
                            rows[r, sl] + eab * lwb[0, sl] + lwb[1, sl], 0.0)
                        plsc.addupdate(acc.at[dl, sl], m)
                return c2
            lax.fori_loop(0, _BLK // 16, row_group, 0)
            return carry
        lax.fori_loop(0, nblk, block, 0)
        pltpu.sync_copy(acc.at[pl.ds(0, _RANGE)],
                        out_hbm.at[pl.ds(w * _RANGE, _RANGE)])

    return sc_layer


_sc_layer_128 = _make_sc_layer()


def _aggregate(z, srcp, dstlp, eap, lw, lb, offs, zeros):
    d = lw.size
    lwb = jnp.stack([lw.reshape(d), lb])
    if d < 128:
        lwb = jnp.pad(lwb, ((0, 0), (0, 128 - d)))
    return _sc_layer_128(z, srcp, dstlp, eap, lwb, offs, zeros)


def _layer_body(z_ref, a_ref, w1_ref, b1_ref, w2_ref, b2_ref, s_ref, t_ref,
                o_ref):
    h = z_ref[...] + a_ref[...]
    u = jnp.maximum(jnp.dot(h, w1_ref[...], preferred_element_type=jnp.float32)
                    + b1_ref[...], 0.0)
    y = jnp.dot(u, w2_ref[...], preferred_element_type=jnp.float32) + b2_ref[...]
    o_ref[...] = jnp.maximum(y, 0.0) * s_ref[...] + t_ref[...]


def _tc_layer(z, a, w1, b1, w2, b2, g, bb, dout_pad=None):
    din, dmid = w1.shape
    dout = w2.shape[1]
    s = (g * _BN_SCALE).reshape(1, dout)
    t = bb.reshape(1, dout)
    w2p, b2p = w2, b2.reshape(1, dout)
    if dout_pad is not None and dout_pad > dout:
        pad = dout_pad - dout
        w2p = jnp.pad(w2, ((0, 0), (0, pad)))
        b2p = jnp.pad(b2p, ((0, 0), (0, pad)))
        s = jnp.pad(s, ((0, 0), (0, pad)))
        t = jnp.pad(t, ((0, 0), (0, pad)))
        dout = dout_pad
    return pl.pallas_call(
        _layer_body,
        grid=(_NB,),
        in_specs=[
            pl.BlockSpec((_RB, din), lambda i: (i, 0)),
            pl.BlockSpec((_RB, din), lambda i: (i, 0)),
            pl.BlockSpec((din, dmid), lambda i: (0, 0)),
            pl.BlockSpec((1, dmid), lambda i: (0, 0)),
            pl.BlockSpec((dmid, dout), lambda i: (0, 0)),
            pl.BlockSpec((1, dout), lambda i: (0, 0)),
            pl.BlockSpec((1, dout), lambda i: (0, 0)),
            pl.BlockSpec((1, dout), lambda i: (0, 0)),
        ],
        out_specs=pl.BlockSpec((_RB, dout), lambda i: (i, 0)),
        out_shape=jax.ShapeDtypeStruct((_N, dout), jnp.float32),
    )(z, a, w1, b1.reshape(1, dmid), w2p, b2p, s, t)


def _final_body(z_ref, a_ref, w1_ref, b1_ref, w2_ref, b2_ref, s_ref, t_ref,
                batch_ref, f1w_ref, f1b_ref, f2w_ref, f2b_ref, f3w_ref,
                f3b_ref, o_ref, acc_ref):
    i = pl.program_id(0)

    @pl.when(i == 0)
    def _():
        acc_ref[...] = jnp.zeros_like(acc_ref)

    h = z_ref[...] + a_ref[...]
    u = jnp.maximum(jnp.dot(h, w1_ref[...], preferred_element_type=jnp.float32)
                    + b1_ref[...], 0.0)
    y = jnp.dot(u, w2_ref[...], preferred_element_type=jnp.float32) + b2_ref[...]
    z3 = jnp.maximum(y, 0.0) * s_ref[...] + t_ref[...]

    b = batch_ref[0, 0, :]
    onehot = (b[:, None] == jax.lax.broadcasted_iota(jnp.int32, (_RB, _G), 1)
              ).astype(jnp.float32)
    acc_ref[...] += jax.lax.dot_general(
        onehot, z3, (((0,), (0,)), ((), ())), preferred_element_type=jnp.float32)

    @pl.when(i == _NB - 1)
    def _():
        p = acc_ref[...]
        h1 = jnp.maximum(jnp.dot(p, f1w_ref[...],
                                 preferred_element_type=jnp.float32)
                         + f1b_ref[...], 0.0)
        h2 = jnp.maximum(jnp.dot(h1, f2w_ref[...],
                                 preferred_element_type=jnp.float32)
                         + f2b_ref[...], 0.0)
        o_ref[...] = jnp.dot(h2, f3w_ref[...],
                             preferred_element_type=jnp.float32) + f3b_ref[...]


def _tc_final(z, a, w1, b1, w2, b2, g, bb, batch, f1w, f1b, f2w, f2b, f3w,
              f3b):
    din, dout = w1.shape
    s = (g * _BN_SCALE).reshape(1, dout)
    t = bb.reshape(1, dout)
    batch3 = batch.reshape(_NB, 1, _RB)
    return pl.pallas_call(
        _final_body,
        grid=(_NB,),
        in_specs=[
            pl.BlockSpec((_RB, din), lambda i: (i, 0)),
            pl.BlockSpec((_RB, din), lambda i: (i, 0)),
            pl.BlockSpec((din, dout), lambda i: (0, 0)),
            pl.BlockSpec((1, dout), lambda i: (0, 0)),
            pl.BlockSpec((dout, dout), lambda i: (0, 0)),
            pl.BlockSpec((1, dout), lambda i: (0, 0)),
            pl.BlockSpec((1, dout), lambda i: (0, 0)),
            pl.BlockSpec((1, dout), lambda i: (0, 0)),
            pl.BlockSpec((1, 1, _RB), lambda i: (i, 0, 0)),
            pl.BlockSpec((dout, 128), lambda i: (0, 0)),
            pl.BlockSpec((1, 128), lambda i: (0, 0)),
            pl.BlockSpec((128, _G), lambda i: (0, 0)),
            pl.BlockSpec((1, _G), lambda i: (0, 0)),
            pl.BlockSpec((_G, _C), lambda i: (0, 0)),
            pl.BlockSpec((1, _C), lambda i: (0, 0)),
        ],
        out_specs=pl.BlockSpec((_G, _C), lambda i: (0, 0)),
        out_shape=jax.ShapeDtypeStruct((_G, _C), jnp.float32),
        scratch_shapes=[pltpu.VMEM((_G, dout), jnp.float32)],
    )(z, a, w1, b1.reshape(1, dout), w2, b2.reshape(1, dout), s, t, batch3,
      f1w, f1b.reshape(1, -1), f2w, f2b.reshape(1, -1), f3w, f3b.reshape(1, -1))


def _partition_edges(src, dst, ea):
    dst_s, src_s, ea_s = lax.sort([dst, src, ea], num_keys=1)
    tgrid = jnp.arange(_NTILES + 1, dtype=jnp.int32) * _RANGE
    bnd = jnp.searchsorted(dst_s, tgrid).astype(jnp.int32)
    counts = bnd[1:] - bnd[:-1]
    nblk = (counts + (_BLK - 1)) // _BLK
    pc = nblk * _BLK
    ps = jnp.concatenate([jnp.zeros((1,), jnp.int32),
                          jnp.cumsum(pc).astype(jnp.int32)])
    slots = jnp.arange(_NBT * _BLK, dtype=jnp.int32)
    st = jnp.minimum(jnp.searchsorted(ps[1:], slots, side='right')
                     .astype(jnp.int32), _NTILES - 1)
    rank = slots - ps[st]
    eidx = jnp.minimum(bnd[st] + rank, _E - 1)
    valid = rank < counts[st]
    srcp = jnp.where(valid, src_s[eidx], 0).reshape(_NBT, _BLK)
    dstlp = jnp.where(valid, dst_s[eidx] - st * _RANGE,
                      _RANGE).reshape(_NBT, _BLK)
    eap = jnp.where(valid, ea_s[eidx], 0.0).reshape(_NBT, _BLK)
    offs = jnp.stack([ps[:_NTILES] // _BLK, nblk], axis=1)
    offs = jnp.pad(offs, ((0, 0), (0, 14)))
    return srcp, dstlp, eap, offs


def kernel(x, edge_index, edge_attr, batch, params):
    p = params
    src, dst = edge_index[0], edge_index[1]
    srcp, dstlp, eap, offs = _partition_edges(src, dst,
                                              edge_attr.reshape(_E))
    z128 = jnp.zeros((_ACC_T, 128), jnp.float32)

    a1 = _aggregate(x, srcp, dstlp, eap, p['lin_e1_w'], p['lin_e1_b'],
                    offs, z128)
    z1 = _tc_layer(x, a1, p['g1_w1'], p['g1_b1'], p['g1_w2'], p['g1_b2'],
                   p['bn1_g'], p['bn1_b'])
    a2 = _aggregate(z1, srcp, dstlp, eap, p['lin_e2_w'], p['lin_e2_b'],
                    offs, z128)
    z2 = _tc_layer(z1, a2, p['g2_w1'], p['g2_b1'], p['g2_w2'], p['g2_b2'],
                   p['bn2_g'], p['bn2_b'], dout_pad=128)
    a3 = _aggregate(z2, srcp, dstlp, eap, p['lin_e3_w'], p['lin_e3_b'],
                    offs, z128)
    w31 = jnp.pad(p['g3_w1'], ((0, 64), (0, 0)))
    out = _tc_final(z2, a3, w31, p['g3_b1'], p['g3_w2'], p['g3_b2'],
                    p['bn3_g'], p['bn3_b'], batch,
                    p['fc1_w'], p['fc1_b'], p['fc2_w'], p['fc2_b'],
                    p['fc3_w'], p['fc3_b'])
    return out


# speedup vs baseline: 1.0059x; 1.0059x over previous
import functools

import jax
import jax.numpy as jnp
from jax import lax
from jax.experimental import pallas as pl
from jax.experimental.pallas import tpu as pltpu
from jax.experimental.pallas import tpu_sc as plsc

_N = 10000
_E = 320000
_G = 64
_C = 10
_RB = 1000
_NB = _N // _RB

_NTILES = 32
_BLK = 128
_IC = 16
_RANGE = 320
_ACC_T = 328
_ACC_N = _NTILES * _RANGE
_BSEG = _IC * _BLK
_NBT = (_E + _NTILES * _BSEG) // _BLK

_BN_SCALE = 1.0 / (1.0 + 1e-5) ** 0.5


def _make_sc_layer():
    d = 128
    mesh = plsc.VectorSubcoreMesh(core_axis_name="c", subcore_axis_name="s")

    @functools.partial(
        pl.kernel,
        out_type=jax.ShapeDtypeStruct((_ACC_N, d), jnp.float32),
        mesh=mesh,
        scratch_types=[
            pltpu.VMEM((_ACC_T, d), jnp.float32),
            pltpu.VMEM((_IC, _BLK), jnp.int32),
            pltpu.VMEM((_IC, _BLK), jnp.int32),
            pltpu.VMEM((_IC, _BLK), jnp.int32),
            pltpu.VMEM((_IC, _BLK), jnp.int32),
            pltpu.VMEM((_IC, _BLK), jnp.float32),
            pltpu.VMEM((_IC, _BLK), jnp.float32),
            pltpu.VMEM((_BLK, d), jnp.float32),
            pltpu.VMEM((_BLK, d), jnp.float32),
            pltpu.VMEM((2, d), jnp.float32),
            pltpu.VMEM((16,), jnp.int32),
            pltpu.SemaphoreType.DMA,
            pltpu.SemaphoreType.DMA,
            pltpu.SemaphoreType.DMA,
            pltpu.SemaphoreType.DMA,
        ],
    )
    def sc_layer(z_hbm, src_hbm, dstl_hbm, ea_hbm, lwb_hbm, offs_hbm,
                 zeros_hbm, out_hbm,
                 acc, src_c0, src_c1, dstl_c0, dstl_c1, ea_c0, ea_c1,
                 rows0, rows1, lwb, offs_v, gsem0, gsem1, isem0, isem1):
        cid = lax.axis_index("c")
        sid = lax.axis_index("s")
        w = cid * 16 + sid
        src_c = (src_c0, src_c1)
        dstl_c = (dstl_c0, dstl_c1)
        ea_c = (ea_c0, ea_c1)
        rows = (rows0, rows1)
        gsem = (gsem0, gsem1)
        isem = (isem0, isem1)

        pltpu.sync_copy(lwb_hbm, lwb)
        pltpu.sync_copy(offs_hbm.at[w], offs_v)
        pltpu.sync_copy(zeros_hbm, acc)
        ov = offs_v[...]
        start = pl.multiple_of(ov[0], _IC)
        nblk = ov[1]
        nch = (nblk + _IC - 1) // _IC

        def fire_ipack(ch, pi):
            g0 = start + ch * _IC
            pltpu.async_copy(src_hbm.at[pl.ds(g0, _IC)], src_c[pi], isem[pi])
            pltpu.async_copy(dstl_hbm.at[pl.ds(g0, _IC)], dstl_c[pi],
                             isem[pi])
            pltpu.async_copy(ea_hbm.at[pl.ds(g0, _IC)], ea_c[pi], isem[pi])

        def drain_ipack(pi):
            pltpu.make_async_copy(src_hbm.at[pl.ds(start, _IC)], src_c[pi],
                                  isem[pi]).wait()
            pltpu.make_async_copy(dstl_hbm.at[pl.ds(start, _IC)], dstl_c[pi],
                                  isem[pi]).wait()
            pltpu.make_async_copy(ea_hbm.at[pl.ds(start, _IC)], ea_c[pi],
                                  isem[pi]).wait()

        def fire_gather(cpi, k, bp):
            pltpu.async_copy(z_hbm.at[src_c[cpi].at[k]], rows[bp], gsem[bp])

        def wait_gather(cpi, bp):
            pltpu.make_async_copy(z_hbm.at[src_c[cpi].at[0]], rows[bp],
                                  gsem[bp]).wait()

        def compute_block(cpi, k, bp):
            def row_group(rr, c2):
                eav = ea_c[cpi][k, pl.ds(rr * 16, 16)]
                dstv = dstl_c[cpi][k, pl.ds(rr * 16, 16)]
                for r16 in range(16):
                    r = rr * 16 + r16
                    eab = eav[r16]
                    dl = dstv[r16]
                    for c in range(d // 16):
                        sl = pl.ds(c * 16, 16)
                        m = jnp.maximum(
                            rows[bp][r, sl] + eab * lwb[0, sl] + lwb[1, sl],
                            0.0)
                        plsc.addupdate(acc.at[dl, sl], m)
                return c2
            lax.fori_loop(0, _BLK // 16, row_group, 0)

        @pl.when(nblk > 0)
        def _():
            pltpu.sync_copy(src_hbm.at[pl.ds(start, _IC)], src_c0)
            pltpu.sync_copy(dstl_hbm.at[pl.ds(start, _IC)], dstl_c0)
            pltpu.sync_copy(ea_hbm.at[pl.ds(start, _IC)], ea_c0)
            fire_gather(0, 0, 0)

            @pl.when(nch > 1)
            def _():
                fire_ipack(1, 1)

            def pair(s2, carry):
                for pi in range(2):
                    c = 2 * s2 + pi

                    @pl.when(c < nch)
                    def _():
                        @pl.when(c > 0)
                        def _():
                            drain_ipack(pi)
                            fire_gather(pi, 0, 0)

                        def inner(s, c2):
                            for j in range(2):
                                k = 2 * s + j
                                b = c * _IC + k

                                @pl.when(b < nblk)
                                def _():
                                    if j == 0:
                                        @pl.when(b + 1 < nblk)
                                        def _():
                                            fire_gather(pi, k + 1, 1 - j)
                                    else:
                                        @pl.when((k < _IC - 1)
                                                 & (b + 1 < nblk))
                                        def _():
                                            fire_gather(pi, k + 1, 1 - j)
                                    wait_gather(pi, j)
                                    compute_block(pi, k, j)
                            return c2
                        lax.fori_loop(0, _IC // 2, inner, 0)

                        @pl.when(c + 2 < nch)
                        def _():
                            fire_ipack(c + 2, pi)
                return carry
            lax.fori_loop(0, (nch + 1) // 2, pair, 0)

        pltpu.sync_copy(acc.at[pl.ds(0, _RANGE)],
                        out_hbm.at[pl.ds(w * _RANGE, _RANGE)])

    return sc_layer


_sc_layer_128 = _make_sc_layer()


def _aggregate(z, srcp, dstlp, eap, lw, lb, offs, zeros):
    d = lw.size
    lwb = jnp.stack([lw.reshape(d), lb])
    if d < 128:
        lwb = jnp.pad(lwb, ((0, 0), (0, 128 - d)))
    return _sc_layer_128(z, srcp, dstlp, eap, lwb, offs, zeros)


def _layer_body(z_ref, a_ref, w1_ref, b1_ref, w2_ref, b2_ref, s_ref, t_ref,
                o_ref):
    h = z_ref[...] + a_ref[...]
    u = jnp.maximum(jnp.dot(h, w1_ref[...], preferred_element_type=jnp.float32)
                    + b1_ref[...], 0.0)
    y = jnp.dot(u, w2_ref[...], preferred_element_type=jnp.float32) + b2_ref[...]
    o_ref[...] = jnp.maximum(y, 0.0) * s_ref[...] + t_ref[...]


def _tc_layer(z, a, w1, b1, w2, b2, g, bb, dout_pad=None):
    din, dmid = w1.shape
    dout = w2.shape[1]
    s = (g * _BN_SCALE).reshape(1, dout)
    t = bb.reshape(1, dout)
    w2p, b2p = w2, b2.reshape(1, dout)
    if dout_pad is not None and dout_pad > dout:
        pad = dout_pad - dout
        w2p = jnp.pad(w2, ((0, 0), (0, pad)))
        b2p = jnp.pad(b2p, ((0, 0), (0, pad)))
        s = jnp.pad(s, ((0, 0), (0, pad)))
        t = jnp.pad(t, ((0, 0), (0, pad)))
        dout = dout_pad
    return pl.pallas_call(
        _layer_body,
        grid=(_NB,),
        in_specs=[
            pl.BlockSpec((_RB, din), lambda i: (i, 0)),
            pl.BlockSpec((_RB, din), lambda i: (i, 0)),
            pl.BlockSpec((din, dmid), lambda i: (0, 0)),
            pl.BlockSpec((1, dmid), lambda i: (0, 0)),
            pl.BlockSpec((dmid, dout), lambda i: (0, 0)),
            pl.BlockSpec((1, dout), lambda i: (0, 0)),
            pl.BlockSpec((1, dout), lambda i: (0, 0)),
            pl.BlockSpec((1, dout), lambda i: (0, 0)),
        ],
        out_specs=pl.BlockSpec((_RB, dout), lambda i: (i, 0)),
        out_shape=jax.ShapeDtypeStruct((_N, dout), jnp.float32),
    )(z, a, w1, b1.reshape(1, dmid), w2p, b2p, s, t)


def _final_body(z_ref, a_ref, w1_ref, b1_ref, w2_ref, b2_ref, s_ref, t_ref,
                batch_ref, f1w_ref, f1b_ref, f2w_ref, f2b_ref, f3w_ref,
                f3b_ref, o_ref, acc_ref):
    i = pl.program_id(0)

    @pl.when(i == 0)
    def _():
        acc_ref[...] = jnp.zeros_like(acc_ref)

    h = z_ref[...] + a_ref[...]
    u = jnp.maximum(jnp.dot(h, w1_ref[...], preferred_element_type=jnp.float32)
                    + b1_ref[...], 0.0)
    y = jnp.dot(u, w2_ref[...], preferred_element_type=jnp.float32) + b2_ref[...]
    z3 = jnp.maximum(y, 0.0) * s_ref[...] + t_ref[...]

    b = batch_ref[0, 0, :]
    onehot = (b[:, None] == jax.lax.broadcasted_iota(jnp.int32, (_RB, _G), 1)
              ).astype(jnp.float32)
    acc_ref[...] += jax.lax.dot_general(
        onehot, z3, (((0,), (0,)), ((), ())), preferred_element_type=jnp.float32)

    @pl.when(i == _NB - 1)
    def _():
        p = acc_ref[...]
        h1 = jnp.maximum(jnp.dot(p, f1w_ref[...],
                                 preferred_element_type=jnp.float32)
                         + f1b_ref[...], 0.0)
        h2 = jnp.maximum(jnp.dot(h1, f2w_ref[...],
                                 preferred_element_type=jnp.float32)
                         + f2b_ref[...], 0.0)
        o_ref[...] = jnp.dot(h2, f3w_ref[...],
                             preferred_element_type=jnp.float32) + f3b_ref[...]


def _tc_final(z, a, w1, b1, w2, b2, g, bb, batch, f1w, f1b, f2w, f2b, f3w,
              f3b):
    din, dout = w1.shape
    s = (g * _BN_SCALE).reshape(1, dout)
    t = bb.reshape(1, dout)
    batch3 = batch.reshape(_NB, 1, _RB)
    return pl.pallas_call(
        _final_body,
        grid=(_NB,),
        in_specs=[
            pl.BlockSpec((_RB, din), lambda i: (i, 0)),
            pl.BlockSpec((_RB, din), lambda i: (i, 0)),
            pl.BlockSpec((din, dout), lambda i: (0, 0)),
            pl.BlockSpec((1, dout), lambda i: (0, 0)),
            pl.BlockSpec((dout, dout), lambda i: (0, 0)),
            pl.BlockSpec((1, dout), lambda i: (0, 0)),
            pl.BlockSpec((1, dout), lambda i: (0, 0)),
            pl.BlockSpec((1, dout), lambda i: (0, 0)),
            pl.BlockSpec((1, 1, _RB), lambda i: (i, 0, 0)),
            pl.BlockSpec((dout, 128), lambda i: (0, 0)),
            pl.BlockSpec((1, 128), lambda i: (0, 0)),
            pl.BlockSpec((128, _G), lambda i: (0, 0)),
            pl.BlockSpec((1, _G), lambda i: (0, 0)),
            pl.BlockSpec((_G, _C), lambda i: (0, 0)),
            pl.BlockSpec((1, _C), lambda i: (0, 0)),
        ],
        out_specs=pl.BlockSpec((_G, _C), lambda i: (0, 0)),
        out_shape=jax.ShapeDtypeStruct((_G, _C), jnp.float32),
        scratch_shapes=[pltpu.VMEM((_G, dout), jnp.float32)],
    )(z, a, w1, b1.reshape(1, dout), w2, b2.reshape(1, dout), s, t, batch3,
      f1w, f1b.reshape(1, -1), f2w, f2b.reshape(1, -1), f3w, f3b.reshape(1, -1))


def _partition_edges(src, dst, ea):
    dst_s, src_s, ea_s = lax.sort([dst, src, ea], num_keys=1)
    tgrid = jnp.arange(_NTILES + 1, dtype=jnp.int32) * _RANGE
    bnd = jnp.searchsorted(dst_s, tgrid).astype(jnp.int32)
    counts = bnd[1:] - bnd[:-1]
    nblk = (counts + (_BLK - 1)) // _BLK
    pc = ((counts + (_BSEG - 1)) // _BSEG) * _BSEG
    ps = jnp.concatenate([jnp.zeros((1,), jnp.int32),
                          jnp.cumsum(pc).astype(jnp.int32)])
    slots = jnp.arange(_NBT * _BLK, dtype=jnp.int32)
    st = jnp.minimum(jnp.searchsorted(ps[1:], slots, side='right')
                     .astype(jnp.int32), _NTILES - 1)
    rank = slots - ps[st]
    eidx = jnp.minimum(bnd[st] + rank, _E - 1)
    valid = rank < counts[st]
    srcp = jnp.where(valid, src_s[eidx], 0).reshape(_NBT, _BLK)
    dstlp = jnp.where(valid, dst_s[eidx] - st * _RANGE,
                      _RANGE).reshape(_NBT, _BLK)
    eap = jnp.where(valid, ea_s[eidx], 0.0).reshape(_NBT, _BLK)
    offs = jnp.stack([ps[:_NTILES] // _BLK, nblk], axis=1)
    offs = jnp.pad(offs, ((0, 0), (0, 14)))
    return srcp, dstlp, eap, offs


def kernel(x, edge_index, edge_attr, batch, params):
    p = params
    src, dst = edge_index[0], edge_index[1]
    srcp, dstlp, eap, offs = _partition_edges(src, dst,
                                              edge_attr.reshape(_E))
    z128 = jnp.zeros((_ACC_T, 128), jnp.float32)

    a1 = _aggregate(x, srcp, dstlp, eap, p['lin_e1_w'], p['lin_e1_b'],
                    offs, z128)
    z1 = _tc_layer(x, a1, p['g1_w1'], p['g1_b1'], p['g1_w2'], p['g1_b2'],
                   p['bn1_g'], p['bn1_b'])
    a2 = _aggregate(z1, srcp, dstlp, eap, p['lin_e2_w'], p['lin_e2_b'],
                    offs, z128)
    z2 = _tc_layer(z1, a2, p['g2_w1'], p['g2_b1'], p['g2_w2'], p['g2_b2'],
                   p['bn2_g'], p['bn2_b'], dout_pad=128)
    a3 = _aggregate(z2, srcp, dstlp, eap, p['lin_e3_w'], p['lin_e3_b'],
                    offs, z128)
    w31 = jnp.pad(p['g3_w1'], ((0, 64), (0, 0)))
    out = _tc_final(z2, a3, w31, p['g3_b1'], p['g3_w2'], p['g3_b2'],
                    p['bn3_g'], p['bn3_b'], batch,
                    p['fc1_w'], p['fc1_b'], p['fc2_w'], p['fc2_b'],
                    p['fc3_w'], p['fc3_b'])
    return out


# speedup vs baseline: 1.3792x; 1.3711x over previous
import functools

import jax
import jax.numpy as jnp
from jax import lax
from jax.experimental import pallas as pl
from jax.experimental.pallas import tpu as pltpu
from jax.experimental.pallas import tpu_sc as plsc

_N = 10000
_E = 320000
_G = 64
_C = 10
_RB = 1000
_NB = _N // _RB

_NTILES = 32
_BLK = 128
_IC = 4
_RANGE = 320
_ACC_T = 328
_ACC_N = _NTILES * _RANGE
_NBLK_TOT = _E // _BLK
_NCH_TOT = _NBLK_TOT // _IC

_BN_SCALE = 1.0 / (1.0 + 1e-5) ** 0.5


def _make_sc_layer():
    d = 128
    mesh = plsc.VectorSubcoreMesh(core_axis_name="c", subcore_axis_name="s")

    @functools.partial(
        pl.kernel,
        out_type=jax.ShapeDtypeStruct((_ACC_N, d), jnp.float32),
        mesh=mesh,
        scratch_types=[
            pltpu.VMEM((_ACC_T, d), jnp.float32),
            pltpu.VMEM((_IC, _BLK), jnp.int32),
            pltpu.VMEM((_IC, _BLK), jnp.int32),
            pltpu.VMEM((_IC, _BLK), jnp.int32),
            pltpu.VMEM((_IC, _BLK), jnp.int32),
            pltpu.VMEM((_IC, _BLK), jnp.float32),
            pltpu.VMEM((_IC, _BLK), jnp.float32),
            pltpu.VMEM((_BLK, d), jnp.float32),
            pltpu.VMEM((_BLK, d), jnp.float32),
            pltpu.VMEM((2, d), jnp.float32),
            pltpu.VMEM((16,), jnp.int32),
            pltpu.SemaphoreType.DMA,
            pltpu.SemaphoreType.DMA,
            pltpu.SemaphoreType.DMA,
            pltpu.SemaphoreType.DMA,
        ],
    )
    def sc_layer(z_hbm, src_hbm, dst_hbm, ea_hbm, lwb_hbm, offs_hbm,
                 zeros_hbm, out_hbm,
                 acc, src_c0, src_c1, dst_c0, dst_c1, ea_c0, ea_c1,
                 rows0, rows1, lwb, offs_v, gsem0, gsem1, isem0, isem1):
        cid = lax.axis_index("c")
        sid = lax.axis_index("s")
        w = cid * 16 + sid
        base = w * _RANGE
        src_c = (src_c0, src_c1)
        dst_c = (dst_c0, dst_c1)
        ea_c = (ea_c0, ea_c1)
        rows = (rows0, rows1)
        gsem = (gsem0, gsem1)
        isem = (isem0, isem1)

        pltpu.sync_copy(lwb_hbm, lwb)
        pltpu.sync_copy(offs_hbm.at[w], offs_v)
        pltpu.sync_copy(zeros_hbm, acc)
        ov = offs_v[...]
        startq = ov[0]
        nblk = ov[1]
        nch = (nblk + _IC - 1) // _IC

        def fire_ipack(ch, pi):
            q = startq + ch
            pltpu.async_copy(src_hbm.at[q], src_c[pi], isem[pi])
            pltpu.async_copy(dst_hbm.at[q], dst_c[pi], isem[pi])
            pltpu.async_copy(ea_hbm.at[q], ea_c[pi], isem[pi])

        def drain_ipack(pi):
            pltpu.make_async_copy(src_hbm.at[startq], src_c[pi],
                                  isem[pi]).wait()
            pltpu.make_async_copy(dst_hbm.at[startq], dst_c[pi],
                                  isem[pi]).wait()
            pltpu.make_async_copy(ea_hbm.at[startq], ea_c[pi],
                                  isem[pi]).wait()

        def fire_gather(cpi, k, bp):
            pltpu.async_copy(z_hbm.at[src_c[cpi].at[k]], rows[bp], gsem[bp])

        def wait_gather(cpi, bp):
            pltpu.make_async_copy(z_hbm.at[src_c[cpi].at[0]], rows[bp],
                                  gsem[bp]).wait()

        def compute_block(cpi, k, bp):
            def row_group(rr, c2):
                eav = ea_c[cpi][k, pl.ds(rr * 16, 16)]
                dlv = dst_c[cpi][k, pl.ds(rr * 16, 16)] - base
                dlv = jnp.where((dlv >= 0) & (dlv < _RANGE), dlv, _RANGE)
                for r16 in range(16):
                    r = rr * 16 + r16
                    eab = eav.at[jnp.full((16,), r16, jnp.int32)].get(
                        mode='promise_in_bounds', indices_are_sorted=True)
                    dl = dlv[r16]
                    for c in range(d // 16):
                        sl = pl.ds(c * 16, 16)
                        m = jnp.maximum(
                            rows[bp][r, sl] + eab * lwb[0, sl] + lwb[1, sl],
                            0.0)
                        plsc.addupdate(acc.at[dl, sl], m)
                return c2
            lax.fori_loop(0, _BLK // 16, row_group, 0)

        @pl.when(nblk > 0)
        def _():
            pltpu.sync_copy(src_hbm.at[startq], src_c0)
            pltpu.sync_copy(dst_hbm.at[startq], dst_c0)
            pltpu.sync_copy(ea_hbm.at[startq], ea_c0)
            fire_gather(0, 0, 0)

            @pl.when(nch > 1)
            def _():
                fire_ipack(1, 1)

            def pair(s2, carry):
                for pi in range(2):
                    c = 2 * s2 + pi

                    @pl.when(c < nch)
                    def _():
                        @pl.when(c > 0)
                        def _():
                            drain_ipack(pi)
                            fire_gather(pi, 0, 0)

                        def inner(s, c2):
                            for j in range(2):
                                k = 2 * s + j
                                b = c * _IC + k

                                @pl.when(b < nblk)
                                def _():
                                    if j == 0:
                                        @pl.when(b + 1 < nblk)
                                        def _():
                                            fire_gather(pi, k + 1, 1 - j)
                                    else:
                                        @pl.when((k < _IC - 1)
                                                 & (b + 1 < nblk))
                                        def _():
                                            fire_gather(pi, k + 1, 1 - j)
                                    wait_gather(pi, j)
                                    compute_block(pi, k, j)
                            return c2
                        lax.fori_loop(0, _IC // 2, inner, 0)

                        @pl.when(c + 2 < nch)
                        def _():
                            fire_ipack(c + 2, pi)
                return carry
            lax.fori_loop(0, (nch + 1) // 2, pair, 0)

        pltpu.sync_copy(acc.at[pl.ds(0, _RANGE)],
                        out_hbm.at[pl.ds(w * _RANGE, _RANGE)])

    return sc_layer


_sc_layer_128 = _make_sc_layer()


def _aggregate(z, srcp, dstlp, eap, lw, lb, offs, zeros):
    d = lw.size
    lwb = jnp.stack([lw.reshape(d), lb])
    if d < 128:
        lwb = jnp.pad(lwb, ((0, 0), (0, 128 - d)))
    return _sc_layer_128(z, srcp, dstlp, eap, lwb, offs, zeros)


def _layer_body(z_ref, a_ref, w1_ref, b1_ref, w2_ref, b2_ref, s_ref, t_ref,
                o_ref):
    h = z_ref[...] + a_ref[...]
    u = jnp.maximum(jnp.dot(h, w1_ref[...], preferred_element_type=jnp.float32)
                    + b1_ref[...], 0.0)
    y = jnp.dot(u, w2_ref[...], preferred_element_type=jnp.float32) + b2_ref[...]
    o_ref[...] = jnp.maximum(y, 0.0) * s_ref[...] + t_ref[...]


def _tc_layer(z, a, w1, b1, w2, b2, g, bb, dout_pad=None):
    din, dmid = w1.shape
    dout = w2.shape[1]
    s = (g * _BN_SCALE).reshape(1, dout)
    t = bb.reshape(1, dout)
    w2p, b2p = w2, b2.reshape(1, dout)
    if dout_pad is not None and dout_pad > dout:
        pad = dout_pad - dout
        w2p = jnp.pad(w2, ((0, 0), (0, pad)))
        b2p = jnp.pad(b2p, ((0, 0), (0, pad)))
        s = jnp.pad(s, ((0, 0), (0, pad)))
        t = jnp.pad(t, ((0, 0), (0, pad)))
        dout = dout_pad
    return pl.pallas_call(
        _layer_body,
        grid=(_NB,),
        in_specs=[
            pl.BlockSpec((_RB, din), lambda i: (i, 0)),
            pl.BlockSpec((_RB, din), lambda i: (i, 0)),
            pl.BlockSpec((din, dmid), lambda i: (0, 0)),
            pl.BlockSpec((1, dmid), lambda i: (0, 0)),
            pl.BlockSpec((dmid, dout), lambda i: (0, 0)),
            pl.BlockSpec((1, dout), lambda i: (0, 0)),
            pl.BlockSpec((1, dout), lambda i: (0, 0)),
            pl.BlockSpec((1, dout), lambda i: (0, 0)),
        ],
        out_specs=pl.BlockSpec((_RB, dout), lambda i: (i, 0)),
        out_shape=jax.ShapeDtypeStruct((_N, dout), jnp.float32),
    )(z, a, w1, b1.reshape(1, dmid), w2p, b2p, s, t)


def _final_body(z_ref, a_ref, w1_ref, b1_ref, w2_ref, b2_ref, s_ref, t_ref,
                batch_ref, f1w_ref, f1b_ref, f2w_ref, f2b_ref, f3w_ref,
                f3b_ref, o_ref, acc_ref):
    i = pl.program_id(0)

    @pl.when(i == 0)
    def _():
        acc_ref[...] = jnp.zeros_like(acc_ref)

    h = z_ref[...] + a_ref[...]
    u = jnp.maximum(jnp.dot(h, w1_ref[...], preferred_element_type=jnp.float32)
                    + b1_ref[...], 0.0)
    y = jnp.dot(u, w2_ref[...], preferred_element_type=jnp.float32) + b2_ref[...]
    z3 = jnp.maximum(y, 0.0) * s_ref[...] + t_ref[...]

    b = batch_ref[0, 0, :]
    onehot = (b[:, None] == jax.lax.broadcasted_iota(jnp.int32, (_RB, _G), 1)
              ).astype(jnp.float32)
    acc_ref[...] += jax.lax.dot_general(
        onehot, z3, (((0,), (0,)), ((), ())), preferred_element_type=jnp.float32)

    @pl.when(i == _NB - 1)
    def _():
        p = acc_ref[...]
        h1 = jnp.maximum(jnp.dot(p, f1w_ref[...],
                                 preferred_element_type=jnp.float32)
                         + f1b_ref[...], 0.0)
        h2 = jnp.maximum(jnp.dot(h1, f2w_ref[...],
                                 preferred_element_type=jnp.float32)
                         + f2b_ref[...], 0.0)
        o_ref[...] = jnp.dot(h2, f3w_ref[...],
                             preferred_element_type=jnp.float32) + f3b_ref[...]


def _tc_final(z, a, w1, b1, w2, b2, g, bb, batch, f1w, f1b, f2w, f2b, f3w,
              f3b):
    din, dout = w1.shape
    s = (g * _BN_SCALE).reshape(1, dout)
    t = bb.reshape(1, dout)
    batch3 = batch.reshape(_NB, 1, _RB)
    return pl.pallas_call(
        _final_body,
        grid=(_NB,),
        in_specs=[
            pl.BlockSpec((_RB, din), lambda i: (i, 0)),
            pl.BlockSpec((_RB, din), lambda i: (i, 0)),
            pl.BlockSpec((din, dout), lambda i: (0, 0)),
            pl.BlockSpec((1, dout), lambda i: (0, 0)),
            pl.BlockSpec((dout, dout), lambda i: (0, 0)),
            pl.BlockSpec((1, dout), lambda i: (0, 0)),
            pl.BlockSpec((1, dout), lambda i: (0, 0)),
            pl.BlockSpec((1, dout), lambda i: (0, 0)),
            pl.BlockSpec((1, 1, _RB), lambda i: (i, 0, 0)),
            pl.BlockSpec((dout, 128), lambda i: (0, 0)),
            pl.BlockSpec((1, 128), lambda i: (0, 0)),
            pl.BlockSpec((128, _G), lambda i: (0, 0)),
            pl.BlockSpec((1, _G), lambda i: (0, 0)),
            pl.BlockSpec((_G, _C), lambda i: (0, 0)),
            pl.BlockSpec((1, _C), lambda i: (0, 0)),
        ],
        out_specs=pl.BlockSpec((_G, _C), lambda i: (0, 0)),
        out_shape=jax.ShapeDtypeStruct((_G, _C), jnp.float32),
        scratch_shapes=[pltpu.VMEM((_G, dout), jnp.float32)],
    )(z, a, w1, b1.reshape(1, dout), w2, b2.reshape(1, dout), s, t, batch3,
      f1w, f1b.reshape(1, -1), f2w, f2b.reshape(1, -1), f3w, f3b.reshape(1, -1))


def _partition_edges(src, dst, ea):
    dst_s, src_s, ea_s = lax.sort([dst, src, ea], num_keys=1)
    tgrid = jnp.arange(_NTILES + 1, dtype=jnp.int32) * _RANGE
    bnd = jnp.searchsorted(dst_s, tgrid).astype(jnp.int32)
    startb = (bnd[:-1] // _BLK) // _IC * _IC
    endb = (bnd[1:] + (_BLK - 1)) // _BLK
    nblk = jnp.maximum(endb - startb, 0)
    offs = jnp.stack([startb // _IC, nblk], axis=1)
    offs = jnp.pad(offs, ((0, 0), (0, 14)))
    srcp = src_s.reshape(_NCH_TOT, _IC, _BLK)
    dstp = dst_s.reshape(_NCH_TOT, _IC, _BLK)
    eap = ea_s.reshape(_NCH_TOT, _IC, _BLK)
    return srcp, dstp, eap, offs


def kernel(x, edge_index, edge_attr, batch, params):
    p = params
    src, dst = edge_index[0], edge_index[1]
    srcp, dstlp, eap, offs = _partition_edges(src, dst,
                                              edge_attr.reshape(_E))
    z128 = jnp.zeros((_ACC_T, 128), jnp.float32)

    a1 = _aggregate(x, srcp, dstlp, eap, p['lin_e1_w'], p['lin_e1_b'],
                    offs, z128)
    z1 = _tc_layer(x, a1, p['g1_w1'], p['g1_b1'], p['g1_w2'], p['g1_b2'],
                   p['bn1_g'], p['bn1_b'])
    a2 = _aggregate(z1, srcp, dstlp, eap, p['lin_e2_w'], p['lin_e2_b'],
                    offs, z128)
    z2 = _tc_layer(z1, a2, p['g2_w1'], p['g2_b1'], p['g2_w2'], p['g2_b2'],
                   p['bn2_g'], p['bn2_b'], dout_pad=128)
    a3 = _aggregate(z2, srcp, dstlp, eap, p['lin_e3_w'], p['lin_e3_b'],
                    offs, z128)
    w31 = jnp.pad(p['g3_w1'], ((0, 64), (0, 0)))
    out = _tc_final(z2, a3, w31, p['g3_b1'], p['g3_w2'], p['g3_b2'],
                    p['bn3_g'], p['bn3_b'], batch,
                    p['fc1_w'], p['fc1_b'], p['fc2_w'], p['fc2_b'],
                    p['fc3_w'], p['fc3_b'])
    return out


# speedup vs baseline: 3.3840x; 2.4537x over previous
import functools

import jax
import jax.numpy as jnp
from jax import lax
from jax.experimental import pallas as pl
from jax.experimental.pallas import tpu as pltpu
from jax.experimental.pallas import tpu_sc as plsc

_N = 10000
_E = 320000
_G = 64
_C = 10
_RB = 1000
_NB = _N // _RB

_NTILES = 32
_BLK = 128
_IC = 4
_RANGE = 320
_ACC_T = 328
_ACC_N = _NTILES * _RANGE
_NBLK_TOT = _E // _BLK
_NCH_TOT = _NBLK_TOT // _IC

_BN_SCALE = 1.0 / (1.0 + 1e-5) ** 0.5


def _make_sc_layer():
    d = 128
    mesh = plsc.VectorSubcoreMesh(core_axis_name="c", subcore_axis_name="s")

    @functools.partial(
        pl.kernel,
        out_type=jax.ShapeDtypeStruct((_ACC_N, d), jnp.float32),
        mesh=mesh,
        scratch_types=[
            pltpu.VMEM((_ACC_T, d), jnp.float32),
            pltpu.VMEM((_IC, _BLK), jnp.int32),
            pltpu.VMEM((_IC, _BLK), jnp.int32),
            pltpu.VMEM((_IC, _BLK), jnp.int32),
            pltpu.VMEM((_IC, _BLK), jnp.int32),
            pltpu.VMEM((_IC, _BLK), jnp.float32),
            pltpu.VMEM((_IC, _BLK), jnp.float32),
            pltpu.VMEM((_BLK, d), jnp.float32),
            pltpu.VMEM((_BLK, d), jnp.float32),
            pltpu.VMEM((2, d), jnp.float32),
            pltpu.VMEM((16,), jnp.int32),
            pltpu.SemaphoreType.DMA,
            pltpu.SemaphoreType.DMA,
            pltpu.SemaphoreType.DMA,
            pltpu.SemaphoreType.DMA,
        ],
    )
    def sc_layer(z_hbm, src_hbm, dst_hbm, ea_hbm, lwb_hbm, offs_hbm,
                 zeros_hbm, out_hbm,
                 acc, src_c0, src_c1, dst_c0, dst_c1, ea_c0, ea_c1,
                 rows0, rows1, lwb, offs_v, gsem0, gsem1, isem0, isem1):
        cid = lax.axis_index("c")
        sid = lax.axis_index("s")
        w = cid * 16 + sid
        base = w * _RANGE
        src_c = (src_c0, src_c1)
        dst_c = (dst_c0, dst_c1)
        ea_c = (ea_c0, ea_c1)
        rows = (rows0, rows1)
        gsem = (gsem0, gsem1)
        isem = (isem0, isem1)

        pltpu.sync_copy(lwb_hbm, lwb)
        pltpu.sync_copy(offs_hbm.at[w], offs_v)
        pltpu.sync_copy(zeros_hbm, acc)
        ov = offs_v[...]
        startq = ov[0]
        nblk = ov[1]
        nch = (nblk + _IC - 1) // _IC

        def fire_ipack(ch, pi):
            q = startq + ch
            pltpu.async_copy(src_hbm.at[q], src_c[pi], isem[pi])
            pltpu.async_copy(dst_hbm.at[q], dst_c[pi], isem[pi])
            pltpu.async_copy(ea_hbm.at[q], ea_c[pi], isem[pi])

        def drain_ipack(pi):
            pltpu.make_async_copy(src_hbm.at[startq], src_c[pi],
                                  isem[pi]).wait()
            pltpu.make_async_copy(dst_hbm.at[startq], dst_c[pi],
                                  isem[pi]).wait()
            pltpu.make_async_copy(ea_hbm.at[startq], ea_c[pi],
                                  isem[pi]).wait()

        def fire_gather(cpi, k, bp):
            pltpu.async_copy(z_hbm.at[src_c[cpi].at[k]], rows[bp], gsem[bp])

        def wait_gather(cpi, bp):
            pltpu.make_async_copy(z_hbm.at[src_c[cpi].at[0]], rows[bp],
                                  gsem[bp]).wait()

        def compute_block(cpi, k, bp):
            def row_group(rr, c2):
                eav = ea_c[cpi][k, pl.ds(rr * 16, 16)]
                dlv = dst_c[cpi][k, pl.ds(rr * 16, 16)] - base
                dlv = jnp.where((dlv >= 0) & (dlv < _RANGE), dlv, _RANGE)
                for q in range(4):
                    ms = []
                    for r4 in range(4):
                        r16 = q * 4 + r4
                        r = rr * 16 + r16
                        eab = eav.at[jnp.full((16,), r16, jnp.int32)].get(
                            mode='promise_in_bounds',
                            indices_are_sorted=True)
                        for c in range(d // 16):
                            sl = pl.ds(c * 16, 16)
                            ms.append(jnp.maximum(
                                rows[bp][r, sl] + eab * lwb[0, sl]
                                + lwb[1, sl], 0.0))
                    i = 0
                    for r4 in range(4):
                        dl = dlv[q * 4 + r4]
                        for c in range(d // 16):
                            plsc.addupdate(acc.at[dl, pl.ds(c * 16, 16)],
                                           ms[i])
                            i += 1
                return c2
            lax.fori_loop(0, _BLK // 16, row_group, 0)

        @pl.when(nblk > 0)
        def _():
            pltpu.sync_copy(src_hbm.at[startq], src_c0)
            pltpu.sync_copy(dst_hbm.at[startq], dst_c0)
            pltpu.sync_copy(ea_hbm.at[startq], ea_c0)
            fire_gather(0, 0, 0)

            @pl.when(nch > 1)
            def _():
                fire_ipack(1, 1)

            def pair(s2, carry):
                for pi in range(2):
                    c = 2 * s2 + pi

                    @pl.when(c < nch)
                    def _():
                        @pl.when(c > 0)
                        def _():
                            drain_ipack(pi)
                            fire_gather(pi, 0, 0)

                        def inner(s, c2):
                            for j in range(2):
                                k = 2 * s + j
                                b = c * _IC + k

                                @pl.when(b < nblk)
                                def _():
                                    if j == 0:
                                        @pl.when(b + 1 < nblk)
                                        def _():
                                            fire_gather(pi, k + 1, 1 - j)
                                    else:
                                        @pl.when((k < _IC - 1)
                                                 & (b + 1 < nblk))
                                        def _():
                                            fire_gather(pi, k + 1, 1 - j)
                                    wait_gather(pi, j)
                                    compute_block(pi, k, j)
                            return c2
                        lax.fori_loop(0, _IC // 2, inner, 0)

                        @pl.when(c + 2 < nch)
                        def _():
                            fire_ipack(c + 2, pi)
                return carry
            lax.fori_loop(0, (nch + 1) // 2, pair, 0)

        pltpu.sync_copy(acc.at[pl.ds(0, _RANGE)],
                        out_hbm.at[pl.ds(w * _RANGE, _RANGE)])

    return sc_layer


_sc_layer_128 = _make_sc_layer()


def _aggregate(z, srcp, dstlp, eap, lw, lb, offs, zeros):
    d = lw.size
    lwb = jnp.stack([lw.reshape(d), lb])
    if d < 128:
        lwb = jnp.pad(lwb, ((0, 0), (0, 128 - d)))
    return _sc_layer_128(z, srcp, dstlp, eap, lwb, offs, zeros)


def _layer_body(z_ref, a_ref, w1_ref, b1_ref, w2_ref, b2_ref, s_ref, t_ref,
                o_ref):
    h = z_ref[...] + a_ref[...]
    u = jnp.maximum(jnp.dot(h, w1_ref[...], preferred_element_type=jnp.float32)
                    + b1_ref[...], 0.0)
    y = jnp.dot(u, w2_ref[...], preferred_element_type=jnp.float32) + b2_ref[...]
    o_ref[...] = jnp.maximum(y, 0.0) * s_ref[...] + t_ref[...]


def _tc_layer(z, a, w1, b1, w2, b2, g, bb, dout_pad=None):
    din, dmid = w1.shape
    dout = w2.shape[1]
    s = (g * _BN_SCALE).reshape(1, dout)
    t = bb.reshape(1, dout)
    w2p, b2p = w2, b2.reshape(1, dout)
    if dout_pad is not None and dout_pad > dout:
        pad = dout_pad - dout
        w2p = jnp.pad(w2, ((0, 0), (0, pad)))
        b2p = jnp.pad(b2p, ((0, 0), (0, pad)))
        s = jnp.pad(s, ((0, 0), (0, pad)))
        t = jnp.pad(t, ((0, 0), (0, pad)))
        dout = dout_pad
    return pl.pallas_call(
        _layer_body,
        grid=(_NB,),
        in_specs=[
            pl.BlockSpec((_RB, din), lambda i: (i, 0)),
            pl.BlockSpec((_RB, din), lambda i: (i, 0)),
            pl.BlockSpec((din, dmid), lambda i: (0, 0)),
            pl.BlockSpec((1, dmid), lambda i: (0, 0)),
            pl.BlockSpec((dmid, dout), lambda i: (0, 0)),
            pl.BlockSpec((1, dout), lambda i: (0, 0)),
            pl.BlockSpec((1, dout), lambda i: (0, 0)),
            pl.BlockSpec((1, dout), lambda i: (0, 0)),
        ],
        out_specs=pl.BlockSpec((_RB, dout), lambda i: (i, 0)),
        out_shape=jax.ShapeDtypeStruct((_N, dout), jnp.float32),
    )(z, a, w1, b1.reshape(1, dmid), w2p, b2p, s, t)


def _final_body(z_ref, a_ref, w1_ref, b1_ref, w2_ref, b2_ref, s_ref, t_ref,
                batch_ref, f1w_ref, f1b_ref, f2w_ref, f2b_ref, f3w_ref,
                f3b_ref, o_ref, acc_ref):
    i = pl.program_id(0)

    @pl.when(i == 0)
    def _():
        acc_ref[...] = jnp.zeros_like(acc_ref)

    h = z_ref[...] + a_ref[...]
    u = jnp.maximum(jnp.dot(h, w1_ref[...], preferred_element_type=jnp.float32)
                    + b1_ref[...], 0.0)
    y = jnp.dot(u, w2_ref[...], preferred_element_type=jnp.float32) + b2_ref[...]
    z3 = jnp.maximum(y, 0.0) * s_ref[...] + t_ref[...]

    b = batch_ref[0, 0, :]
    onehot = (b[:, None] == jax.lax.broadcasted_iota(jnp.int32, (_RB, _G), 1)
              ).astype(jnp.float32)
    acc_ref[...] += jax.lax.dot_general(
        onehot, z3, (((0,), (0,)), ((), ())), preferred_element_type=jnp.float32)

    @pl.when(i == _NB - 1)
    def _():
        p = acc_ref[...]
        h1 = jnp.maximum(jnp.dot(p, f1w_ref[...],
                                 preferred_element_type=jnp.float32)
                         + f1b_ref[...], 0.0)
        h2 = jnp.maximum(jnp.dot(h1, f2w_ref[...],
                                 preferred_element_type=jnp.float32)
                         + f2b_ref[...], 0.0)
        o_ref[...] = jnp.dot(h2, f3w_ref[...],
                             preferred_element_type=jnp.float32) + f3b_ref[...]


def _tc_final(z, a, w1, b1, w2, b2, g, bb, batch, f1w, f1b, f2w, f2b, f3w,
              f3b):
    din, dout = w1.shape
    s = (g * _BN_SCALE).reshape(1, dout)
    t = bb.reshape(1, dout)
    batch3 = batch.reshape(_NB, 1, _RB)
    return pl.pallas_call(
        _final_body,
        grid=(_NB,),
        in_specs=[
            pl.BlockSpec((_RB, din), lambda i: (i, 0)),
            pl.BlockSpec((_RB, din), lambda i: (i, 0)),
            pl.BlockSpec((din, dout), lambda i: (0, 0)),
            pl.BlockSpec((1, dout), lambda i: (0, 0)),
            pl.BlockSpec((dout, dout), lambda i: (0, 0)),
            pl.BlockSpec((1, dout), lambda i: (0, 0)),
            pl.BlockSpec((1, dout), lambda i: (0, 0)),
            pl.BlockSpec((1, dout), lambda i: (0, 0)),
            pl.BlockSpec((1, 1, _RB), lambda i: (i, 0, 0)),
            pl.BlockSpec((dout, 128), lambda i: (0, 0)),
            pl.BlockSpec((1, 128), lambda i: (0, 0)),
            pl.BlockSpec((128, _G), lambda i: (0, 0)),
            pl.BlockSpec((1, _G), lambda i: (0, 0)),
            pl.BlockSpec((_G, _C), lambda i: (0, 0)),
            pl.BlockSpec((1, _C), lambda i: (0, 0)),
        ],
        out_specs=pl.BlockSpec((_G, _C), lambda i: (0, 0)),
        out_shape=jax.ShapeDtypeStruct((_G, _C), jnp.float32),
        scratch_shapes=[pltpu.VMEM((_G, dout), jnp.float32)],
    )(z, a, w1, b1.reshape(1, dout), w2, b2.reshape(1, dout), s, t, batch3,
      f1w, f1b.reshape(1, -1), f2w, f2b.reshape(1, -1), f3w, f3b.reshape(1, -1))


def _partition_edges(src, dst, ea):
    dst_s, src_s, ea_s = lax.sort([dst, src, ea], num_keys=1)
    tgrid = jnp.arange(_NTILES + 1, dtype=jnp.int32) * _RANGE
    bnd = jnp.searchsorted(dst_s, tgrid).astype(jnp.int32)
    startb = (bnd[:-1] // _BLK) // _IC * _IC
    endb = (bnd[1:] + (_BLK - 1)) // _BLK
    nblk = jnp.maximum(endb - startb, 0)
    offs = jnp.stack([startb // _IC, nblk], axis=1)
    offs = jnp.pad(offs, ((0, 0), (0, 14)))
    srcp = src_s.reshape(_NCH_TOT, _IC, _BLK)
    dstp = dst_s.reshape(_NCH_TOT, _IC, _BLK)
    eap = ea_s.reshape(_NCH_TOT, _IC, _BLK)
    return srcp, dstp, eap, offs


def kernel(x, edge_index, edge_attr, batch, params):
    p = params
    src, dst = edge_index[0], edge_index[1]
    srcp, dstlp, eap, offs = _partition_edges(src, dst,
                                              edge_attr.reshape(_E))
    z128 = jnp.zeros((_ACC_T, 128), jnp.float32)

    a1 = _aggregate(x, srcp, dstlp, eap, p['lin_e1_w'], p['lin_e1_b'],
                    offs, z128)
    z1 = _tc_layer(x, a1, p['g1_w1'], p['g1_b1'], p['g1_w2'], p['g1_b2'],
                   p['bn1_g'], p['bn1_b'])
    a2 = _aggregate(z1, srcp, dstlp, eap, p['lin_e2_w'], p['lin_e2_b'],
                    offs, z128)
    z2 = _tc_layer(z1, a2, p['g2_w1'], p['g2_b1'], p['g2_w2'], p['g2_b2'],
                   p['bn2_g'], p['bn2_b'], dout_pad=128)
    a3 = _aggregate(z2, srcp, dstlp, eap, p['lin_e3_w'], p['lin_e3_b'],
                    offs, z128)
    w31 = jnp.pad(p['g3_w1'], ((0, 64), (0, 0)))
    out = _tc_final(z2, a3, w31, p['g3_b1'], p['g3_w2'], p['g3_b2'],
                    p['bn3_g'], p['bn3_b'], batch,
                    p['fc1_w'], p['fc1_b'], p['fc2_w'], p['fc2_b'],
                    p['fc3_w'], p['fc3_b'])
    return out
